# baseline (speedup 1.0000x reference)
import functools

import jax
import jax.numpy as jnp
from jax import lax
from jax.experimental import pallas as pl
from jax.experimental.pallas import tpu as pltpu
from jax.experimental.pallas import tpu_sc as plsc

_BATCH = 16384
_D = 64
_NW = 32
_CHUNK = 128
_ROWS_PER_W = _BATCH // _NW
_CHUNKS_PER_W = _ROWS_PER_W // _CHUNK


def _sc_gather_body(U_hbm, M_hbm, uidx_hbm, midx_hbm, ue_hbm, me_hbm,
                    uidx_v, midx_v, urows_v, mrows_v, sem_u, sem_m):
    wid = lax.axis_index("s") * 2 + lax.axis_index("c")
    row0 = wid * _CHUNKS_PER_W
    pltpu.sync_copy(uidx_hbm.at[pl.ds(row0, _CHUNKS_PER_W)], uidx_v)
    pltpu.sync_copy(midx_hbm.at[pl.ds(row0, _CHUNKS_PER_W)], midx_v)
    copies = []
    for j in range(_CHUNKS_PER_W):
        dst = urows_v.at[pl.ds(j * _CHUNK, _CHUNK)]
        copies.append(pltpu.async_copy(U_hbm.at[uidx_v.at[j]], dst, sem_u))
        dst = mrows_v.at[pl.ds(j * _CHUNK, _CHUNK)]
        copies.append(pltpu.async_copy(M_hbm.at[midx_v.at[j]], dst, sem_m))
    for c in copies:
        c.wait()
    base = wid * _ROWS_PER_W
    pltpu.sync_copy(urows_v, ue_hbm.at[pl.ds(base, _ROWS_PER_W)])
    pltpu.sync_copy(mrows_v, me_hbm.at[pl.ds(base, _ROWS_PER_W)])


def _sc_gather(U, M, users2d, movies2d):
    mesh = plsc.VectorSubcoreMesh(core_axis_name="c", subcore_axis_name="s")
    k = functools.partial(
        pl.kernel,
        mesh=mesh,
        compiler_params=pltpu.CompilerParams(use_tc_tiling_on_sc=False),
        out_type=[
            jax.ShapeDtypeStruct((_BATCH, _D), jnp.float32),
            jax.ShapeDtypeStruct((_BATCH, _D), jnp.float32),
        ],
        scratch_types=[
            pltpu.VMEM((_CHUNKS_PER_W, _CHUNK), jnp.int32),
            pltpu.VMEM((_CHUNKS_PER_W, _CHUNK), jnp.int32),
            pltpu.VMEM((_ROWS_PER_W, _D), jnp.float32),
            pltpu.VMEM((_ROWS_PER_W, _D), jnp.float32),
            pltpu.SemaphoreType.DMA,
            pltpu.SemaphoreType.DMA,
        ],
    )(_sc_gather_body)
    return k(U, M, users2d, movies2d)


def _mlp_body(ue_ref, me_ref, g_ref, w1_ref, b1_ref, w2_ref, b2_ref,
              wf_ref, bf_ref, out_ref):
    w1 = w1_ref[...]
    h = ue_ref[...] @ w1[0:_D, :]
    h += me_ref[...] @ w1[_D:2 * _D, :]
    h += g_ref[...] @ w1[2 * _D:, :]
    h = jnp.maximum(h + b1_ref[...], 0.0)
    h = jnp.maximum(h @ w2_ref[...] + b2_ref[...], 0.0)
    o = jnp.sum(h * wf_ref[...], axis=1, keepdims=True) + bf_ref[...]
    out_ref[...] = 1.0 / (1.0 + jnp.exp(-o))


def _mlp(ue, me, genres, W1, b1, W2, b2, Wf, bf):
    bb = 2048
    grid = (_BATCH // bb,)
    n_in = 2 * _D + genres.shape[1]
    h1, h2 = W1.shape[1], W2.shape[1]
    return pl.pallas_call(
        _mlp_body,
        grid=grid,
        in_specs=[
            pl.BlockSpec((bb, _D), lambda i: (i, 0)),
            pl.BlockSpec((bb, _D), lambda i: (i, 0)),
            pl.BlockSpec((bb, genres.shape[1]), lambda i: (i, 0)),
            pl.BlockSpec((n_in, h1), lambda i: (0, 0)),
            pl.BlockSpec((1, h1), lambda i: (0, 0)),
            pl.BlockSpec((h1, h2), lambda i: (0, 0)),
            pl.BlockSpec((1, h2), lambda i: (0, 0)),
            pl.BlockSpec((1, h2), lambda i: (0, 0)),
            pl.BlockSpec((1, 1), lambda i: (0, 0)),
        ],
        out_specs=pl.BlockSpec((bb, 1), lambda i: (i, 0)),
        out_shape=jax.ShapeDtypeStruct((_BATCH, 1), jnp.float32),
    )(ue, me, genres, W1, b1, W2, b2, Wf, bf)


def kernel(users, movies, genres, U, M, W1, b1, W2, b2, Wf, bf):
    users2d = users.astype(jnp.int32).reshape(_BATCH // _CHUNK, _CHUNK)
    movies2d = movies.astype(jnp.int32).reshape(_BATCH // _CHUNK, _CHUNK)
    ue, me = _sc_gather(U, M, users2d, movies2d)
    return _mlp(ue, me, genres,
                W1, b1.reshape(1, -1), W2, b2.reshape(1, -1),
                Wf.reshape(1, -1), bf.reshape(1, 1))
